# baseline (speedup 1.0000x reference)
import functools

import jax
import jax.numpy as jnp
from jax import lax
from jax.experimental import pallas as pl
from jax.experimental.pallas import tpu as pltpu
from jax.experimental.pallas import tpu_sc as plsc

_T = 32
_LEN_M = 8
_CN = 8
_NCHUNK = 1250
_NW = 32


def _make_sc_blend(Tn, Nn, Fn):
    mesh = plsc.VectorSubcoreMesh(core_axis_name="c", subcore_axis_name="s")

    @functools.partial(
        pl.kernel,
        mesh=mesh,
        out_type=jax.ShapeDtypeStruct((Tn, Nn, Fn), jnp.float32),
        scratch_types=[
            pltpu.VMEM((_T * _LEN_M,), jnp.float32),
            pltpu.VMEM((_T, _CN, Fn), jnp.float32),
            pltpu.VMEM((_T, _CN, Fn), jnp.float32),
        ],
    )
    def sc_blend(mpad_hbm, x_hbm, out_hbm, e_v, in_v, out_v):
        wid = lax.axis_index("s") * 2 + lax.axis_index("c")
        pltpu.sync_copy(mpad_hbm, e_v)
        p = []
        for i in range(_T * _LEN_M // 16):
            ev = jnp.exp(e_v[pl.ds(16 * i, 16)])
            for half in range(2):
                s = ev[8 * half]
                for j in range(1, _LEN_M):
                    s = s + ev[8 * half + j]
                pv = ev / jnp.full((16,), s, jnp.float32)
                p.append([pv[8 * half + j] for j in range(_LEN_M)])

        ngroups = Fn // 16

        def chunk_body(i, carry):
            ci = wid + _NW * i
            n0 = ci * _CN
            pltpu.sync_copy(x_hbm.at[:, pl.ds(n0, _CN), :], in_v)

            def node_body(n, ncarry):
                for k in range(ngroups):
                    window = []
                    for t in range(_T):
                        xv = in_v[t, n, pl.ds(16 * k, 16)]
                        window.append(xv)
                        start = max(0, t - (_LEN_M - 1))
                        acc = p[t][0] * window[start]
                        for j in range(1, t - start + 1):
                            acc = acc + p[t][j] * window[start + j]
                        out_v[t, n, pl.ds(16 * k, 16)] = acc
                return ncarry

            lax.fori_loop(0, _CN, node_body, 0)
            pltpu.sync_copy(out_v, out_hbm.at[:, pl.ds(n0, _CN), :])
            return carry

        niter = (_NCHUNK - wid + _NW - 1) // _NW
        lax.fori_loop(0, niter, chunk_body, 0)

    return sc_blend


def _assemble_mpad(M):
    rows = []
    for t in range(_T):
        src = M[t] if t < _LEN_M else M[t - 1]
        row = src[0]
        if row.shape[0] < _LEN_M:
            row = jnp.pad(row, (0, _LEN_M - row.shape[0]),
                          constant_values=-1e30)
        rows.append(row)
    return jnp.stack(rows, axis=0).reshape(_T * _LEN_M)


@jax.jit
def kernel(X, M):
    Tn, Nn, Fn = X.shape
    mpad = _assemble_mpad(M)
    return _make_sc_blend(Tn, Nn, Fn)(mpad, X)


# speedup vs baseline: 1.7404x; 1.7404x over previous
import functools

import jax
import jax.numpy as jnp
from jax.experimental import pallas as pl

_T = 32
_LEN_M = 8
_BN = 200


def _softmax_p(mpad):
    logits = mpad - jnp.max(mpad, axis=1, keepdims=True)
    e = jnp.exp(logits)
    return e / jnp.sum(e, axis=1, keepdims=True)


def _blend_kernel(mpad_ref, x_ref, o_ref):
    p = _softmax_p(mpad_ref[...])
    for t in range(_T):
        start = max(0, t - (_LEN_M - 1))
        acc = p[t, 0] * x_ref[start]
        for j in range(1, t - start + 1):
            acc = acc + p[t, j] * x_ref[start + j]
        o_ref[t] = acc


def _assemble_mpad(M):
    rows = []
    for t in range(_T):
        src = M[t] if t < _LEN_M else M[t - 1]
        row = src[0]
        if row.shape[0] < _LEN_M:
            row = jnp.pad(row, (0, _LEN_M - row.shape[0]),
                          constant_values=-1e30)
        rows.append(row)
    return jnp.stack(rows, axis=0)


@functools.partial(jax.jit, static_argnums=())
def kernel(X, M):
    Tn, Nn, Fn = X.shape
    mpad = _assemble_mpad(M)
    grid = Nn // _BN
    out = pl.pallas_call(
        _blend_kernel,
        grid=(grid,),
        in_specs=[
            pl.BlockSpec((_T, _LEN_M), lambda i: (0, 0)),
            pl.BlockSpec((_T, _BN, Fn), lambda i: (0, i, 0)),
        ],
        out_specs=pl.BlockSpec((_T, _BN, Fn), lambda i: (0, i, 0)),
        out_shape=jax.ShapeDtypeStruct((Tn, Nn, Fn), jnp.float32),
    )(mpad, X)
    return out
